# Initial kernel scaffold; baseline (speedup 1.0000x reference)
import jax
import jax.numpy as jnp
from jax.experimental import pallas as pl


def kernel(x, edge_index, W1, b1, W2, b2):
    raise NotImplementedError("write your pallas kernel here")


# speedup vs baseline: 25.1252x; 25.1252x over previous
import functools

import jax
import jax.numpy as jnp
from jax import lax
from jax.experimental import pallas as pl
from jax.experimental.pallas import tpu as pltpu
from jax.experimental.pallas import tpu_sc as plsc

N = 10000
E = 320000
F_IN = 128
HID = 128
C_OUT = 64

NPAD = 10240
NW = 32
CW = 128
NCHUNK = 79
EPT = NCHUNK * CW
EPAD = NW * EPT
RPT = NPAD // 16
BLK = 512
GRID = NPAD // BLK

_mesh = plsc.VectorSubcoreMesh(core_axis_name="c", subcore_axis_name="s")


@functools.partial(
    pl.kernel,
    out_type=jax.ShapeDtypeStruct((NW, NPAD), jnp.float32),
    mesh=_mesh,
    scratch_types=[
        pltpu.VMEM((EPT,), jnp.int32),
        pltpu.VMEM((NPAD,), jnp.float32),
    ],
    compiler_params=pltpu.CompilerParams(needs_layout_passes=False),
)
def _deg_kernel(dst_hbm, out_hbm, dst_v, deg_v):
    c = lax.axis_index("c")
    s = lax.axis_index("s")
    wid = s * 2 + c
    pltpu.sync_copy(dst_hbm.at[wid], dst_v)
    zeros = jnp.zeros((16,), jnp.float32)
    ones = jnp.ones((16,), jnp.float32)

    def zero_body(i, _):
        deg_v[pl.ds(i * 16, 16)] = zeros
        return 0

    lax.fori_loop(0, NPAD // 16, zero_body, 0)

    def body(i, _):
        idx = dst_v[pl.ds(i * 16, 16)]
        plsc.addupdate_scatter(deg_v, [idx], ones)
        return 0

    lax.fori_loop(0, EPT // 16, body, 0)
    pltpu.sync_copy(deg_v, out_hbm.at[wid])


def _make_spmm(d):
    @functools.partial(
        pl.kernel,
        out_type=jax.ShapeDtypeStruct((2 * NPAD, d), jnp.float32),
        mesh=_mesh,
        scratch_types=[
            pltpu.VMEM((EPT,), jnp.int32),
            pltpu.VMEM((NCHUNK, CW), jnp.int32),
            pltpu.VMEM((CW, d), jnp.float32),
            pltpu.VMEM_SHARED((NPAD, d), jnp.float32),
            pltpu.SemaphoreType.DMA,
        ],
        compiler_params=pltpu.CompilerParams(
            use_tc_tiling_on_sc=(d % 128 == 0)),
    )
    def spmm(y_hbm, src_hbm, dst_hbm, zeros_hbm, out_hbm,
             src_v, dst_v, rows_v, acc_sh, sem):
        c = lax.axis_index("c")
        s = lax.axis_index("s")
        wid = s * 2 + c
        pltpu.sync_copy(zeros_hbm.at[pl.ds(s * RPT, RPT)],
                        acc_sh.at[pl.ds(s * RPT, RPT)])
        pltpu.sync_copy(src_hbm.at[wid], src_v)
        pltpu.sync_copy(dst_hbm.at[wid], dst_v)
        plsc.subcore_barrier()

        def body(j, _):
            gat = pltpu.async_copy(
                y_hbm.at[src_v.at[pl.ds(j * CW, CW)]], rows_v, sem)
            gat.wait()
            pltpu.sync_copy(rows_v, acc_sh.at[dst_v.at[j]], add=True)
            return 0

        lax.fori_loop(0, NCHUNK, body, 0)
        plsc.subcore_barrier()
        pltpu.sync_copy(acc_sh.at[pl.ds(s * RPT, RPT)],
                        out_hbm.at[pl.ds(c * NPAD + s * RPT, RPT)])

    return spmm


_spmm_hid = _make_spmm(HID)
_spmm_out = _make_spmm(C_OUT)


def _dinv_of(deg_blk):
    deg = jnp.sum(deg_blk, axis=0)
    return lax.rsqrt(deg + 1.0)


def _y1_body(deg_ref, x_ref, w1_ref, y_ref):
    i = pl.program_id(0)
    dinv = _dinv_of(deg_ref[...])
    xw = jnp.dot(x_ref[...], w1_ref[...], preferred_element_type=jnp.float32)
    rowid = lax.broadcasted_iota(jnp.int32, (BLK, HID), 0) + i * BLK
    y_ref[...] = jnp.where(rowid < N, xw * dinv[:, None], 0.0)


def _y2_body(deg_ref, s1a_ref, s1b_ref, y1_ref, b1_ref, w2_ref, y2_ref):
    i = pl.program_id(0)
    dinv = _dinv_of(deg_ref[...])
    pre = (s1a_ref[...] + s1b_ref[...] + y1_ref[...]) * dinv[:, None]
    h = jnp.maximum(pre + b1_ref[...], 0.0)
    rowid = lax.broadcasted_iota(jnp.int32, (BLK, HID), 0) + i * BLK
    h = jnp.where(rowid < N, h, 0.0)
    xw2 = jnp.dot(h, w2_ref[...], preferred_element_type=jnp.float32)
    y2_ref[...] = xw2 * dinv[:, None]


def _out_body(deg_ref, s2a_ref, s2b_ref, y2_ref, b2_ref, out_ref):
    dinv = _dinv_of(deg_ref[...])
    acc = (s2a_ref[...] + s2b_ref[...] + y2_ref[...]) * dinv[:, None]
    out_ref[...] = acc + b2_ref[...]


def _deg_spec():
    return pl.BlockSpec((NW, BLK), lambda i: (0, i))


def _rows(d):
    return pl.BlockSpec((BLK, d), lambda i: (i, 0))


def _rows_hi(d):
    return pl.BlockSpec((BLK, d), lambda i: (i + GRID, 0))


def _full(shape):
    return pl.BlockSpec(shape, lambda i: (0,) * len(shape))


def kernel(x, edge_index, W1, b1, W2, b2):
    src = edge_index[0]
    dst = edge_index[1]
    pad_idx = N + (jnp.arange(EPAD - E, dtype=jnp.int32) % (NPAD - N))
    src_p = jnp.concatenate([src, pad_idx]).reshape(NW, EPT)
    dst_p = jnp.concatenate([dst, pad_idx]).reshape(NW, NCHUNK, CW)
    x_p = jnp.concatenate(
        [x, jnp.zeros((NPAD - N, F_IN), jnp.float32)], axis=0)
    zeros_hid = jnp.zeros((NPAD, HID), jnp.float32)
    zeros_out = jnp.zeros((NPAD, C_OUT), jnp.float32)

    deg_parts = _deg_kernel(dst_p.reshape(NW, EPT))

    y1 = pl.pallas_call(
        _y1_body,
        grid=(GRID,),
        in_specs=[_deg_spec(), _rows(F_IN), _full((F_IN, HID))],
        out_specs=_rows(HID),
        out_shape=jax.ShapeDtypeStruct((NPAD, HID), jnp.float32),
    )(deg_parts, x_p, W1)

    s1 = _spmm_hid(y1, src_p, dst_p, zeros_hid)

    y2 = pl.pallas_call(
        _y2_body,
        grid=(GRID,),
        in_specs=[_deg_spec(), _rows(HID), _rows_hi(HID), _rows(HID),
                  _full((1, HID)), _full((HID, C_OUT))],
        out_specs=_rows(C_OUT),
        out_shape=jax.ShapeDtypeStruct((NPAD, C_OUT), jnp.float32),
    )(deg_parts, s1, s1, y1, b1.reshape(1, HID), W2)

    s2 = _spmm_out(y2, src_p, dst_p, zeros_out)

    out = pl.pallas_call(
        _out_body,
        grid=(GRID,),
        in_specs=[_deg_spec(), _rows(C_OUT), _rows_hi(C_OUT), _rows(C_OUT),
                  _full((1, C_OUT))],
        out_specs=_rows(C_OUT),
        out_shape=jax.ShapeDtypeStruct((NPAD, C_OUT), jnp.float32),
    )(deg_parts, s2, s2, y2, b2.reshape(1, C_OUT))

    return out[:N]


# speedup vs baseline: 34.4067x; 1.3694x over previous
import functools

import jax
import jax.numpy as jnp
from jax import lax
from jax.experimental import pallas as pl
from jax.experimental.pallas import tpu as pltpu
from jax.experimental.pallas import tpu_sc as plsc

N = 10000
E = 320000
F_IN = 128
HID = 128
C_OUT = 64

NPAD = 10240
NW = 32
CW = 128
NCHUNK = 80
EPT = NCHUNK * CW
EPAD = NW * EPT
RPT = NPAD // 16
BLK = 512
GRID = NPAD // BLK

_mesh = plsc.VectorSubcoreMesh(core_axis_name="c", subcore_axis_name="s")


@functools.partial(
    pl.kernel,
    out_type=jax.ShapeDtypeStruct((NW, NPAD), jnp.float32),
    mesh=_mesh,
    scratch_types=[
        pltpu.VMEM((EPT,), jnp.int32),
        pltpu.VMEM((NPAD,), jnp.float32),
    ],
    compiler_params=pltpu.CompilerParams(needs_layout_passes=False),
)
def _deg_kernel(dst_hbm, out_hbm, dst_v, deg_v):
    c = lax.axis_index("c")
    s = lax.axis_index("s")
    wid = s * 2 + c
    pltpu.sync_copy(dst_hbm.at[wid], dst_v)
    zeros = jnp.zeros((16,), jnp.float32)
    ones = jnp.ones((16,), jnp.float32)

    def zero_body(i, _):
        deg_v[pl.ds(i * 16, 16)] = zeros
        return 0

    lax.fori_loop(0, NPAD // 16, zero_body, 0)

    def body(i, _):
        idx = dst_v[pl.ds(i * 16, 16)]
        plsc.addupdate_scatter(deg_v, [idx], ones)
        return 0

    lax.fori_loop(0, EPT // 16, body, 0)
    pltpu.sync_copy(deg_v, out_hbm.at[wid])


G = 16
NG = NCHUNK // G


def _make_spmm(d):
    @functools.partial(
        pl.kernel,
        out_type=jax.ShapeDtypeStruct((2 * NPAD, d), jnp.float32),
        mesh=_mesh,
        scratch_types=[
            pltpu.VMEM((EPT,), jnp.int32),
            pltpu.VMEM((G, CW), jnp.int32),
            pltpu.VMEM((G, CW), jnp.int32),
            pltpu.VMEM((CW, d), jnp.float32),
            pltpu.VMEM((CW, d), jnp.float32),
            pltpu.VMEM_SHARED((NPAD, d), jnp.float32),
            pltpu.SemaphoreType.DMA,
            pltpu.SemaphoreType.DMA,
            pltpu.SemaphoreType.DMA,
            pltpu.SemaphoreType.DMA,
        ],
        compiler_params=pltpu.CompilerParams(
            use_tc_tiling_on_sc=(d % 128 == 0)),
    )
    def spmm(y_hbm, src_hbm, dst_hbm, zeros_hbm, out_hbm,
             src_v, dst_a, dst_b, rows_a, rows_b, acc_sh,
             sem_a, sem_b, sem_ia, sem_ib):
        c = lax.axis_index("c")
        s = lax.axis_index("s")
        wid = s * 2 + c
        pltpu.sync_copy(zeros_hbm.at[pl.ds(s * RPT, RPT)],
                        acc_sh.at[pl.ds(s * RPT, RPT)])
        pltpu.sync_copy(src_hbm.at[wid], src_v)

        bufs = (rows_a, rows_b)
        sems = (sem_a, sem_b)
        dbufs = (dst_a, dst_b)
        isems = (sem_ia, sem_ib)

        def src_at(j):
            return y_hbm.at[src_v.at[pl.ds(j * CW, CW)]]

        def gather(j, b):
            pltpu.async_copy(src_at(j), bufs[b], sems[b])

        def dst_group(g):
            return dst_hbm.at[wid, pl.ds(g * G, G)]

        plsc.subcore_barrier()
        pltpu.async_copy(dst_group(0), dbufs[0], isems[0])
        gather(0, 0)
        gather(1, 1)

        for g in range(NG):
            gb = g % 2
            pltpu.make_async_copy(dst_group(g), dbufs[gb], isems[gb]).wait()
            if g + 1 < NG:
                pltpu.async_copy(dst_group(g + 1), dbufs[1 - gb],
                                 isems[1 - gb])

            def pair(i, _):
                for b in range(2):
                    jj = g * G + i * 2 + b
                    pltpu.make_async_copy(src_at(jj), bufs[b],
                                          sems[b]).wait()
                    pltpu.sync_copy(bufs[b],
                                    acc_sh.at[dbufs[gb].at[i * 2 + b]],
                                    add=True)
                    gather(jj + 2, b)
                return 0

            if g + 1 < NG:
                lax.fori_loop(0, G // 2, pair, 0)
            else:
                lax.fori_loop(0, G // 2 - 1, pair, 0)
                for b in range(2):
                    jj = NCHUNK - 2 + b
                    pltpu.make_async_copy(src_at(jj), bufs[b],
                                          sems[b]).wait()
                    pltpu.sync_copy(bufs[b],
                                    acc_sh.at[dbufs[gb].at[G - 2 + b]],
                                    add=True)
        plsc.subcore_barrier()
        pltpu.sync_copy(acc_sh.at[pl.ds(s * RPT, RPT)],
                        out_hbm.at[pl.ds(c * NPAD + s * RPT, RPT)])

    return spmm


_spmm_hid = _make_spmm(HID)
_spmm_out = _make_spmm(C_OUT)


def _dinv_of(deg_blk):
    deg = jnp.sum(deg_blk, axis=0)
    return lax.rsqrt(deg + 1.0)


def _y1_body(deg_ref, x_ref, w1_ref, y_ref):
    i = pl.program_id(0)
    dinv = _dinv_of(deg_ref[...])
    xw = jnp.dot(x_ref[...], w1_ref[...], preferred_element_type=jnp.float32)
    rowid = lax.broadcasted_iota(jnp.int32, (BLK, HID), 0) + i * BLK
    y_ref[...] = jnp.where(rowid < N, xw * dinv[:, None], 0.0)


def _y2_body(deg_ref, s1a_ref, s1b_ref, y1_ref, b1_ref, w2_ref, y2_ref):
    i = pl.program_id(0)
    dinv = _dinv_of(deg_ref[...])
    pre = (s1a_ref[...] + s1b_ref[...] + y1_ref[...]) * dinv[:, None]
    h = jnp.maximum(pre + b1_ref[...], 0.0)
    rowid = lax.broadcasted_iota(jnp.int32, (BLK, HID), 0) + i * BLK
    h = jnp.where(rowid < N, h, 0.0)
    xw2 = jnp.dot(h, w2_ref[...], preferred_element_type=jnp.float32)
    y2_ref[...] = xw2 * dinv[:, None]


def _out_body(deg_ref, s2a_ref, s2b_ref, y2_ref, b2_ref, out_ref):
    dinv = _dinv_of(deg_ref[...])
    acc = (s2a_ref[...] + s2b_ref[...] + y2_ref[...]) * dinv[:, None]
    out_ref[...] = acc + b2_ref[...]


def _deg_spec():
    return pl.BlockSpec((NW, BLK), lambda i: (0, i))


def _rows(d):
    return pl.BlockSpec((BLK, d), lambda i: (i, 0))


def _rows_hi(d):
    return pl.BlockSpec((BLK, d), lambda i: (i + GRID, 0))


def _full(shape):
    return pl.BlockSpec(shape, lambda i: (0,) * len(shape))


def kernel(x, edge_index, W1, b1, W2, b2):
    src = edge_index[0]
    dst = edge_index[1]
    pad_idx = N + (jnp.arange(EPAD - E, dtype=jnp.int32) % (NPAD - N))
    src_p = jnp.concatenate([src, pad_idx]).reshape(NW, EPT)
    dst_p = jnp.concatenate([dst, pad_idx]).reshape(NW, NCHUNK, CW)
    x_p = jnp.concatenate(
        [x, jnp.zeros((NPAD - N, F_IN), jnp.float32)], axis=0)
    zeros_hid = jnp.zeros((NPAD, HID), jnp.float32)
    zeros_out = jnp.zeros((NPAD, C_OUT), jnp.float32)

    deg_parts = _deg_kernel(dst_p.reshape(NW, EPT))

    y1 = pl.pallas_call(
        _y1_body,
        grid=(GRID,),
        in_specs=[_deg_spec(), _rows(F_IN), _full((F_IN, HID))],
        out_specs=_rows(HID),
        out_shape=jax.ShapeDtypeStruct((NPAD, HID), jnp.float32),
    )(deg_parts, x_p, W1)

    s1 = _spmm_hid(y1, src_p, dst_p, zeros_hid)

    y2 = pl.pallas_call(
        _y2_body,
        grid=(GRID,),
        in_specs=[_deg_spec(), _rows(HID), _rows_hi(HID), _rows(HID),
                  _full((1, HID)), _full((HID, C_OUT))],
        out_specs=_rows(C_OUT),
        out_shape=jax.ShapeDtypeStruct((NPAD, C_OUT), jnp.float32),
    )(deg_parts, s1, s1, y1, b1.reshape(1, HID), W2)

    s2 = _spmm_out(y2, src_p, dst_p, zeros_out)

    out = pl.pallas_call(
        _out_body,
        grid=(GRID,),
        in_specs=[_deg_spec(), _rows(C_OUT), _rows_hi(C_OUT), _rows(C_OUT),
                  _full((1, C_OUT))],
        out_specs=_rows(C_OUT),
        out_shape=jax.ShapeDtypeStruct((NPAD, C_OUT), jnp.float32),
    )(deg_parts, s2, s2, y2, b2.reshape(1, C_OUT))

    return out[:N]


# speedup vs baseline: 41.2798x; 1.1998x over previous
import functools

import jax
import jax.numpy as jnp
from jax import lax
from jax.experimental import pallas as pl
from jax.experimental.pallas import tpu as pltpu
from jax.experimental.pallas import tpu_sc as plsc

N = 10000
E = 320000
F_IN = 128
HID = 128
C_OUT = 64

NPAD = 10240
NW = 32
CW = 128
NCHUNK = 80
EPT = NCHUNK * CW
EPAD = NW * EPT
RPT = NPAD // 16
BLK = 512
GRID = NPAD // BLK
G = 16
NG = NCHUNK // G
NBUF = 4

_mesh = plsc.VectorSubcoreMesh(core_axis_name="c", subcore_axis_name="s")


@functools.partial(
    pl.kernel,
    out_type=jax.ShapeDtypeStruct((NW, NPAD), jnp.float32),
    mesh=_mesh,
    scratch_types=[
        pltpu.VMEM((EPT,), jnp.int32),
        pltpu.VMEM((NPAD,), jnp.float32),
    ],
    compiler_params=pltpu.CompilerParams(needs_layout_passes=False),
)
def _deg_kernel(dst_hbm, out_hbm, dst_v, deg_v):
    c = lax.axis_index("c")
    s = lax.axis_index("s")
    wid = s * 2 + c
    pltpu.sync_copy(dst_hbm.at[wid], dst_v)
    zeros = jnp.zeros((16,), jnp.float32)
    ones = jnp.ones((16,), jnp.float32)

    def zero_body(i, _):
        deg_v[pl.ds(i * 16, 16)] = zeros
        return 0

    lax.fori_loop(0, NPAD // 16, zero_body, 0)

    def body(i, _):
        idx = dst_v[pl.ds(i * 16, 16)]
        plsc.addupdate_scatter(deg_v, [idx], ones)
        return 0

    lax.fori_loop(0, EPT // 16, body, 0)
    pltpu.sync_copy(deg_v, out_hbm.at[wid])


def _make_spmm(d):
    @functools.partial(
        pl.kernel,
        out_type=jax.ShapeDtypeStruct((2 * NPAD, d), jnp.bfloat16),
        mesh=_mesh,
        scratch_types=[
            pltpu.VMEM((EPT,), jnp.int32),
            pltpu.VMEM((G, CW), jnp.int32),
            pltpu.VMEM((G, CW), jnp.int32),
            [pltpu.VMEM((CW, d), jnp.bfloat16) for _ in range(NBUF)],
            pltpu.VMEM_SHARED((NPAD, d), jnp.bfloat16),
            [pltpu.SemaphoreType.DMA for _ in range(NBUF)],
            pltpu.SemaphoreType.DMA,
            pltpu.SemaphoreType.DMA,
        ],
        compiler_params=pltpu.CompilerParams(use_tc_tiling_on_sc=False),
    )
    def spmm(y_hbm, src_hbm, dst_hbm, zeros_hbm, out_hbm,
             src_v, dst_a, dst_b, bufs, acc_sh, sems, sem_ia, sem_ib):
        c = lax.axis_index("c")
        s = lax.axis_index("s")
        wid = s * 2 + c
        pltpu.sync_copy(zeros_hbm.at[pl.ds(s * RPT, RPT)],
                        acc_sh.at[pl.ds(s * RPT, RPT)])
        pltpu.sync_copy(src_hbm.at[wid], src_v)

        dbufs = (dst_a, dst_b)
        isems = (sem_ia, sem_ib)

        def src_at(j):
            return y_hbm.at[src_v.at[pl.ds(j * CW, CW)]]

        def gather(j, b):
            pltpu.async_copy(src_at(j), bufs[b], sems[b])

        def dst_group(g):
            return dst_hbm.at[wid, pl.ds(g * G, G)]

        plsc.subcore_barrier()
        pltpu.async_copy(dst_group(0), dbufs[0], isems[0])
        for b in range(NBUF):
            gather(b, b)

        for g in range(NG):
            gb = g % 2
            pltpu.make_async_copy(dst_group(g), dbufs[gb], isems[gb]).wait()
            if g + 1 < NG:
                pltpu.async_copy(dst_group(g + 1), dbufs[1 - gb],
                                 isems[1 - gb])

            def quad(i, _, drain=False):
                for b in range(NBUF):
                    jj = g * G + i * NBUF + b
                    pltpu.make_async_copy(src_at(jj), bufs[b],
                                          sems[b]).wait()
                    pltpu.sync_copy(bufs[b],
                                    acc_sh.at[dbufs[gb].at[i * NBUF + b]],
                                    add=True)
                    if not drain:
                        gather(jj + NBUF, b)
                return 0

            if g + 1 < NG:
                lax.fori_loop(0, G // NBUF, quad, 0)
            else:
                lax.fori_loop(0, G // NBUF - 1, quad, 0)
                quad(G // NBUF - 1, 0, drain=True)
        plsc.subcore_barrier()
        pltpu.sync_copy(acc_sh.at[pl.ds(s * RPT, RPT)],
                        out_hbm.at[pl.ds(c * NPAD + s * RPT, RPT)])

    return spmm


_spmm_hid = _make_spmm(HID)
_spmm_out = _make_spmm(C_OUT)


def _dinv_of(deg_blk):
    deg = jnp.sum(deg_blk, axis=0)
    return lax.rsqrt(deg + 1.0)


def _mm_body(x_ref, w1_ref, xw_ref):
    xw_ref[...] = jnp.dot(x_ref[...], w1_ref[...],
                          preferred_element_type=jnp.float32)


def _y1_body(deg_ref, xw_ref, y_ref):
    i = pl.program_id(0)
    dinv = _dinv_of(deg_ref[...])
    rowid = lax.broadcasted_iota(jnp.int32, (BLK, HID), 0) + i * BLK
    y = jnp.where(rowid < N, xw_ref[...] * dinv[:, None], 0.0)
    y_ref[...] = y.astype(jnp.bfloat16)


def _y2_body(deg_ref, s1a_ref, s1b_ref, y1_ref, b1_ref, w2_ref, y2_ref):
    i = pl.program_id(0)
    dinv = _dinv_of(deg_ref[...])
    agg = (s1a_ref[...].astype(jnp.float32) +
           s1b_ref[...].astype(jnp.float32) +
           y1_ref[...].astype(jnp.float32))
    h = jnp.maximum(agg * dinv[:, None] + b1_ref[...], 0.0)
    rowid = lax.broadcasted_iota(jnp.int32, (BLK, HID), 0) + i * BLK
    h = jnp.where(rowid < N, h, 0.0)
    xw2 = jnp.dot(h, w2_ref[...], preferred_element_type=jnp.float32)
    y2_ref[...] = (xw2 * dinv[:, None]).astype(jnp.bfloat16)


def _out_body(deg_ref, s2a_ref, s2b_ref, y2_ref, b2_ref, out_ref):
    dinv = _dinv_of(deg_ref[...])
    agg = (s2a_ref[...].astype(jnp.float32) +
           s2b_ref[...].astype(jnp.float32) +
           y2_ref[...].astype(jnp.float32))
    out_ref[...] = agg * dinv[:, None] + b2_ref[...]


def _deg_spec():
    return pl.BlockSpec((NW, BLK), lambda i: (0, i))


def _rows(d):
    return pl.BlockSpec((BLK, d), lambda i: (i, 0))


def _rows_hi(d):
    return pl.BlockSpec((BLK, d), lambda i: (i + GRID, 0))


def _full(shape):
    return pl.BlockSpec(shape, lambda i: (0,) * len(shape))


def kernel(x, edge_index, W1, b1, W2, b2):
    src = edge_index[0]
    dst = edge_index[1]
    pad_idx = N + (jnp.arange(EPAD - E, dtype=jnp.int32) % (NPAD - N))
    src_p = jnp.concatenate([src, pad_idx]).reshape(NW, EPT)
    dst_p = jnp.concatenate([dst, pad_idx]).reshape(NW, NCHUNK, CW)
    x_p = jnp.concatenate(
        [x, jnp.zeros((NPAD - N, F_IN), jnp.float32)], axis=0)
    zeros_hid = jnp.zeros((NPAD, HID), jnp.bfloat16)
    zeros_out = jnp.zeros((NPAD, C_OUT), jnp.bfloat16)

    deg_parts = _deg_kernel(dst_p.reshape(NW, EPT))

    xw1 = pl.pallas_call(
        _mm_body,
        grid=(GRID,),
        in_specs=[_rows(F_IN), _full((F_IN, HID))],
        out_specs=_rows(HID),
        out_shape=jax.ShapeDtypeStruct((NPAD, HID), jnp.float32),
    )(x_p, W1)

    y1 = pl.pallas_call(
        _y1_body,
        grid=(GRID,),
        in_specs=[_deg_spec(), _rows(HID)],
        out_specs=_rows(HID),
        out_shape=jax.ShapeDtypeStruct((NPAD, HID), jnp.bfloat16),
    )(deg_parts, xw1)

    s1 = _spmm_hid(y1, src_p, dst_p, zeros_hid)

    y2 = pl.pallas_call(
        _y2_body,
        grid=(GRID,),
        in_specs=[_deg_spec(), _rows(HID), _rows_hi(HID), _rows(HID),
                  _full((1, HID)), _full((HID, C_OUT))],
        out_specs=_rows(C_OUT),
        out_shape=jax.ShapeDtypeStruct((NPAD, C_OUT), jnp.bfloat16),
    )(deg_parts, s1, s1, y1, b1.reshape(1, HID), W2)

    s2 = _spmm_out(y2, src_p, dst_p, zeros_out)

    out = pl.pallas_call(
        _out_body,
        grid=(GRID,),
        in_specs=[_deg_spec(), _rows(C_OUT), _rows_hi(C_OUT), _rows(C_OUT),
                  _full((1, C_OUT))],
        out_specs=_rows(C_OUT),
        out_shape=jax.ShapeDtypeStruct((NPAD, C_OUT), jnp.float32),
    )(deg_parts, s2, s2, y2, b2.reshape(1, C_OUT))

    return out[:N]


# speedup vs baseline: 43.3892x; 1.0511x over previous
import functools

import jax
import jax.numpy as jnp
from jax import lax
from jax.experimental import pallas as pl
from jax.experimental.pallas import tpu as pltpu
from jax.experimental.pallas import tpu_sc as plsc

N = 10000
E = 320000
F_IN = 128
HID = 128
C_OUT = 64

NPAD = 10240
NW = 32
CW = 128
NCHUNK = 80
EPT = NCHUNK * CW
EPAD = NW * EPT
RPT = NPAD // 16
BLK = 512
GRID = NPAD // BLK
NRING = 8
LAG = 4

_mesh = plsc.VectorSubcoreMesh(core_axis_name="c", subcore_axis_name="s")


@functools.partial(
    pl.kernel,
    out_type=jax.ShapeDtypeStruct((NW, NPAD), jnp.float32),
    mesh=_mesh,
    scratch_types=[
        pltpu.VMEM((EPT,), jnp.int32),
        pltpu.VMEM((NPAD,), jnp.float32),
    ],
    compiler_params=pltpu.CompilerParams(needs_layout_passes=False),
)
def _deg_kernel(dst_hbm, out_hbm, dst_v, deg_v):
    c = lax.axis_index("c")
    s = lax.axis_index("s")
    wid = s * 2 + c
    pltpu.sync_copy(dst_hbm.at[wid], dst_v)
    zeros = jnp.zeros((16,), jnp.float32)
    ones = jnp.ones((16,), jnp.float32)

    def zero_body(i, _):
        for u in range(4):
            deg_v[pl.ds((i * 4 + u) * 16, 16)] = zeros
        return 0

    lax.fori_loop(0, NPAD // 64, zero_body, 0)

    def body(i, _):
        idx = dst_v[pl.ds(i * 16, 16)]
        plsc.addupdate_scatter(deg_v, [idx], ones)
        return 0

    lax.fori_loop(0, EPT // 16, body, 0)
    pltpu.sync_copy(deg_v, out_hbm.at[wid])


def _make_spmm(d):
    @functools.partial(
        pl.kernel,
        out_type=jax.ShapeDtypeStruct((2 * NPAD, d), jnp.bfloat16),
        mesh=_mesh,
        scratch_types=[
            pltpu.VMEM((EPT,), jnp.int32),
            pltpu.VMEM((NCHUNK, CW), jnp.int32),
            [pltpu.VMEM((CW, d), jnp.bfloat16) for _ in range(NRING)],
            pltpu.VMEM_SHARED((NPAD, d), jnp.bfloat16),
            [pltpu.SemaphoreType.DMA for _ in range(NRING)],
            [pltpu.SemaphoreType.DMA for _ in range(NRING)],
        ],
        compiler_params=pltpu.CompilerParams(use_tc_tiling_on_sc=False),
    )
    def spmm(y_hbm, src_hbm, dst_hbm, zeros_hbm, out_hbm,
             src_v, dst_v, bufs, acc_sh, gsems, ssems):
        c = lax.axis_index("c")
        s = lax.axis_index("s")
        wid = s * 2 + c
        pltpu.sync_copy(zeros_hbm.at[pl.ds(s * RPT, RPT)],
                        acc_sh.at[pl.ds(s * RPT, RPT)])
        pltpu.sync_copy(src_hbm.at[wid], src_v)
        pltpu.sync_copy(dst_hbm.at[wid], dst_v)

        def src_at(j):
            return y_hbm.at[src_v.at[pl.ds(j * CW, CW)]]

        def gather(j, b):
            pltpu.async_copy(src_at(j), bufs[b], gsems[b])

        def wait_gather(j, b):
            pltpu.make_async_copy(src_at(j), bufs[b], gsems[b]).wait()

        def scatter(j, b):
            pltpu.async_copy(bufs[b], acc_sh.at[dst_v.at[j]], ssems[b],
                             add=True)

        def wait_scatter(j, b):
            pltpu.make_async_copy(bufs[b], acc_sh.at[dst_v.at[j]],
                                  ssems[b]).wait()

        plsc.subcore_barrier()
        for b in range(LAG):
            gather(b, b)

        for jj in range(NRING):
            b = jj % NRING
            wait_gather(jj, b)
            scatter(jj, b)
            bg = (jj + LAG) % NRING
            if jj + LAG >= NRING:
                wait_scatter(jj + LAG - NRING, bg)
            gather(jj + LAG, bg)

        def body(i, _):
            for b in range(NRING):
                jj = i * NRING + b
                wait_gather(jj, b)
                scatter(jj, b)
                bg = (b + LAG) % NRING
                wait_scatter(jj + LAG - NRING, bg)
                gather(jj + LAG, bg)
            return 0

        lax.fori_loop(1, NCHUNK // NRING - 1, body, 0)

        for b in range(NRING):
            jj = NCHUNK - NRING + b
            wait_gather(jj, b)
            scatter(jj, b)
            if jj + LAG < NCHUNK:
                bg = (b + LAG) % NRING
                wait_scatter(jj + LAG - NRING, bg)
                gather(jj + LAG, bg)
        for b in range(NRING):
            wait_scatter(NCHUNK - NRING + b, b)

        plsc.subcore_barrier()
        pltpu.sync_copy(acc_sh.at[pl.ds(s * RPT, RPT)],
                        out_hbm.at[pl.ds(c * NPAD + s * RPT, RPT)])

    return spmm


_spmm_hid = _make_spmm(HID)
_spmm_out = _make_spmm(C_OUT)


def _dinv_of(deg_blk):
    deg = jnp.sum(deg_blk, axis=0)
    return lax.rsqrt(deg + 1.0)


def _y1_body(deg_ref, x_ref, w1_ref, y_ref):
    i = pl.program_id(0)
    dinv = _dinv_of(deg_ref[...])
    xw = jnp.dot(x_ref[...], w1_ref[...], preferred_element_type=jnp.float32)
    rowid = lax.broadcasted_iota(jnp.int32, (BLK, HID), 0) + i * BLK
    y = jnp.where(rowid < N, xw * dinv[:, None], 0.0)
    y_ref[...] = y.astype(jnp.bfloat16)


def _y2_body(deg_ref, s1_ref, y1_ref, b1_ref, w2_ref, y2_ref):
    i = pl.program_id(0)
    dinv = _dinv_of(deg_ref[...])
    agg = (s1_ref[0].astype(jnp.float32) + s1_ref[1].astype(jnp.float32) +
           y1_ref[...].astype(jnp.float32))
    h = jnp.maximum(agg * dinv[:, None] + b1_ref[...], 0.0)
    rowid = lax.broadcasted_iota(jnp.int32, (BLK, HID), 0) + i * BLK
    h = jnp.where(rowid < N, h, 0.0)
    xw2 = jnp.dot(h, w2_ref[...], preferred_element_type=jnp.float32)
    y2_ref[...] = (xw2 * dinv[:, None]).astype(jnp.bfloat16)


def _out_body(deg_ref, s2_ref, y2_ref, b2_ref, out_ref):
    dinv = _dinv_of(deg_ref[...])
    agg = (s2_ref[0].astype(jnp.float32) + s2_ref[1].astype(jnp.float32) +
           y2_ref[...].astype(jnp.float32))
    out_ref[...] = agg * dinv[:, None] + b2_ref[...]


def _deg_spec():
    return pl.BlockSpec((NW, BLK), lambda i: (0, i))


def _rows(d):
    return pl.BlockSpec((BLK, d), lambda i: (i, 0))


def _parts(d):
    return pl.BlockSpec((2, BLK, d), lambda i: (0, i, 0))


def _full(shape):
    return pl.BlockSpec(shape, lambda i: (0,) * len(shape))


def kernel(x, edge_index, W1, b1, W2, b2):
    src = edge_index[0]
    dst = edge_index[1]
    pad_idx = N + (jnp.arange(EPAD - E, dtype=jnp.int32) % (NPAD - N))
    src_p = jnp.concatenate([src, pad_idx]).reshape(NW, EPT)
    dst_p = jnp.concatenate([dst, pad_idx]).reshape(NW, NCHUNK, CW)
    x_p = jnp.concatenate(
        [x, jnp.zeros((NPAD - N, F_IN), jnp.float32)], axis=0)
    zeros_hid = jnp.zeros((NPAD, HID), jnp.bfloat16)
    zeros_out = jnp.zeros((NPAD, C_OUT), jnp.bfloat16)

    deg_parts = _deg_kernel(dst_p.reshape(NW, EPT))

    y1 = pl.pallas_call(
        _y1_body,
        grid=(GRID,),
        in_specs=[_deg_spec(), _rows(F_IN), _full((F_IN, HID))],
        out_specs=_rows(HID),
        out_shape=jax.ShapeDtypeStruct((NPAD, HID), jnp.bfloat16),
    )(deg_parts, x_p, W1)

    s1 = _spmm_hid(y1, src_p, dst_p, zeros_hid).reshape(2, NPAD, HID)

    y2 = pl.pallas_call(
        _y2_body,
        grid=(GRID,),
        in_specs=[_deg_spec(), _parts(HID), _rows(HID),
                  _full((1, HID)), _full((HID, C_OUT))],
        out_specs=_rows(C_OUT),
        out_shape=jax.ShapeDtypeStruct((NPAD, C_OUT), jnp.bfloat16),
    )(deg_parts, s1, y1, b1.reshape(1, HID), W2)

    s2 = _spmm_out(y2, src_p, dst_p, zeros_out).reshape(2, NPAD, C_OUT)

    out = pl.pallas_call(
        _out_body,
        grid=(GRID,),
        in_specs=[_deg_spec(), _parts(C_OUT), _rows(C_OUT),
                  _full((1, C_OUT))],
        out_specs=_rows(C_OUT),
        out_shape=jax.ShapeDtypeStruct((NPAD, C_OUT), jnp.float32),
    )(deg_parts, s2, y2, b2.reshape(1, C_OUT))

    return out[:N]


# speedup vs baseline: 43.4502x; 1.0014x over previous
import functools

import jax
import jax.numpy as jnp
from jax import lax
from jax.experimental import pallas as pl
from jax.experimental.pallas import tpu as pltpu
from jax.experimental.pallas import tpu_sc as plsc

N = 10000
E = 320000
F_IN = 128
HID = 128
C_OUT = 64

NPAD = 10240
NW = 32
CW = 128
NCHUNK = 80
EPT = NCHUNK * CW
EPAD = NW * EPT
RPT = NPAD // 16
BLK = 512
GRID = NPAD // BLK
NRING = 8
LAG = 4

_mesh = plsc.VectorSubcoreMesh(core_axis_name="c", subcore_axis_name="s")


@functools.partial(
    pl.kernel,
    out_type=jax.ShapeDtypeStruct((NW, NPAD), jnp.float32),
    mesh=_mesh,
    scratch_types=[
        pltpu.VMEM((EPT,), jnp.int32),
        pltpu.VMEM((NPAD,), jnp.float32),
    ],
    compiler_params=pltpu.CompilerParams(needs_layout_passes=False),
)
def _deg_kernel(dst_hbm, out_hbm, dst_v, deg_v):
    c = lax.axis_index("c")
    s = lax.axis_index("s")
    wid = s * 2 + c
    pltpu.sync_copy(dst_hbm.at[wid], dst_v)
    zeros = jnp.zeros((16,), jnp.float32)
    ones = jnp.ones((16,), jnp.float32)

    def zero_body(i, _):
        for u in range(4):
            deg_v[pl.ds((i * 4 + u) * 16, 16)] = zeros
        return 0

    lax.fori_loop(0, NPAD // 64, zero_body, 0)

    def body(i, _):
        idx = dst_v[pl.ds(i * 16, 16)]
        plsc.addupdate_scatter(deg_v, [idx], ones)
        return 0

    lax.fori_loop(0, EPT // 16, body, 0)
    pltpu.sync_copy(deg_v, out_hbm.at[wid])


def _make_spmm(d):
    @functools.partial(
        pl.kernel,
        out_type=jax.ShapeDtypeStruct((2, NPAD, d), jnp.bfloat16),
        mesh=_mesh,
        scratch_types=[
            pltpu.VMEM((EPT,), jnp.int32),
            pltpu.VMEM((NCHUNK, CW), jnp.int32),
            [pltpu.VMEM((CW, d), jnp.bfloat16) for _ in range(NRING)],
            pltpu.VMEM_SHARED((NPAD, d), jnp.bfloat16),
            [pltpu.SemaphoreType.DMA for _ in range(NRING)],
            [pltpu.SemaphoreType.DMA for _ in range(NRING)],
        ],
        compiler_params=pltpu.CompilerParams(use_tc_tiling_on_sc=False),
    )
    def spmm(y_hbm, src_hbm, dst_hbm, zeros_hbm, out_hbm,
             src_v, dst_v, bufs, acc_sh, gsems, ssems):
        c = lax.axis_index("c")
        s = lax.axis_index("s")
        wid = s * 2 + c
        pltpu.sync_copy(zeros_hbm.at[pl.ds(s * RPT, RPT)],
                        acc_sh.at[pl.ds(s * RPT, RPT)])
        pltpu.sync_copy(src_hbm.at[wid], src_v)
        pltpu.sync_copy(dst_hbm.at[wid], dst_v)

        def src_at(j):
            return y_hbm.at[src_v.at[pl.ds(j * CW, CW)]]

        def gather(j, b):
            pltpu.async_copy(src_at(j), bufs[b], gsems[b])

        def wait_gather(j, b):
            pltpu.make_async_copy(src_at(j), bufs[b], gsems[b]).wait()

        def scatter(j, b):
            pltpu.async_copy(bufs[b], acc_sh.at[dst_v.at[j]], ssems[b],
                             add=True)

        def wait_scatter(j, b):
            pltpu.make_async_copy(bufs[b], acc_sh.at[dst_v.at[j]],
                                  ssems[b]).wait()

        plsc.subcore_barrier()
        for b in range(LAG):
            gather(b, b)

        for jj in range(NRING):
            b = jj % NRING
            wait_gather(jj, b)
            scatter(jj, b)
            bg = (jj + LAG) % NRING
            if jj + LAG >= NRING:
                wait_scatter(jj + LAG - NRING, bg)
            gather(jj + LAG, bg)

        def body(i, _):
            for b in range(NRING):
                jj = i * NRING + b
                wait_gather(jj, b)
                scatter(jj, b)
                bg = (b + LAG) % NRING
                wait_scatter(jj + LAG - NRING, bg)
                gather(jj + LAG, bg)
            return 0

        lax.fori_loop(1, NCHUNK // NRING - 1, body, 0)

        for b in range(NRING):
            jj = NCHUNK - NRING + b
            wait_gather(jj, b)
            scatter(jj, b)
            if jj + LAG < NCHUNK:
                bg = (b + LAG) % NRING
                wait_scatter(jj + LAG - NRING, bg)
                gather(jj + LAG, bg)
        for b in range(NRING):
            wait_scatter(NCHUNK - NRING + b, b)

        plsc.subcore_barrier()
        pltpu.sync_copy(acc_sh.at[pl.ds(s * RPT, RPT)],
                        out_hbm.at[c, pl.ds(s * RPT, RPT)])

    return spmm


_spmm_hid = _make_spmm(HID)
_spmm_out = _make_spmm(C_OUT)


def _dinv_of(deg_blk):
    deg = jnp.sum(deg_blk, axis=0)
    return lax.rsqrt(deg + 1.0)


def _y1_body(deg_ref, x_ref, w1_ref, y_ref):
    i = pl.program_id(0)
    dinv = _dinv_of(deg_ref[...])
    xw = jnp.dot(x_ref[...], w1_ref[...], preferred_element_type=jnp.float32)
    rowid = lax.broadcasted_iota(jnp.int32, (BLK, HID), 0) + i * BLK
    y = jnp.where(rowid < N, xw * dinv[:, None], 0.0)
    y_ref[...] = y.astype(jnp.bfloat16)


def _y2_body(deg_ref, s1a_ref, s1b_ref, y1_ref, b1_ref, w2_ref, y2_ref):
    i = pl.program_id(0)
    dinv = _dinv_of(deg_ref[...])
    agg = (s1a_ref[0].astype(jnp.float32) + s1b_ref[0].astype(jnp.float32)
           + y1_ref[...].astype(jnp.float32))
    h = jnp.maximum(agg * dinv[:, None] + b1_ref[...], 0.0)
    rowid = lax.broadcasted_iota(jnp.int32, (BLK, HID), 0) + i * BLK
    h = jnp.where(rowid < N, h, 0.0)
    xw2 = jnp.dot(h, w2_ref[...], preferred_element_type=jnp.float32)
    y2_ref[...] = (xw2 * dinv[:, None]).astype(jnp.bfloat16)


def _out_body(deg_ref, s2a_ref, s2b_ref, y2_ref, b2_ref, out_ref):
    dinv = _dinv_of(deg_ref[...])
    agg = (s2a_ref[0].astype(jnp.float32) + s2b_ref[0].astype(jnp.float32)
           + y2_ref[...].astype(jnp.float32))
    out_ref[...] = agg * dinv[:, None] + b2_ref[...]


def _deg_spec():
    return pl.BlockSpec((NW, BLK), lambda i: (0, i))


def _rows(d):
    return pl.BlockSpec((BLK, d), lambda i: (i, 0))


def _part_a(d):
    return pl.BlockSpec((1, BLK, d), lambda i: (0, i, 0))


def _part_b(d):
    return pl.BlockSpec((1, BLK, d), lambda i: (1, i, 0))


def _full(shape):
    return pl.BlockSpec(shape, lambda i: (0,) * len(shape))


def kernel(x, edge_index, W1, b1, W2, b2):
    src = edge_index[0]
    dst = edge_index[1]
    pad_idx = N + (jnp.arange(EPAD - E, dtype=jnp.int32) % (NPAD - N))
    src_p = jnp.concatenate([src, pad_idx]).reshape(NW, EPT)
    dst_p = jnp.concatenate([dst, pad_idx]).reshape(NW, NCHUNK, CW)
    zeros_hid = jnp.zeros((NPAD, HID), jnp.bfloat16)
    zeros_out = jnp.zeros((NPAD, C_OUT), jnp.bfloat16)

    x_p = jnp.concatenate(
        [x, jnp.zeros((NPAD - N, F_IN), jnp.float32)], axis=0)

    deg_parts = _deg_kernel(dst_p.reshape(NW, EPT))

    y1 = pl.pallas_call(
        _y1_body,
        grid=(GRID,),
        in_specs=[_deg_spec(), _rows(F_IN), _full((F_IN, HID))],
        out_specs=_rows(HID),
        out_shape=jax.ShapeDtypeStruct((NPAD, HID), jnp.bfloat16),
    )(deg_parts, x_p, W1)

    s1 = _spmm_hid(y1, src_p, dst_p, zeros_hid)

    y2 = pl.pallas_call(
        _y2_body,
        grid=(GRID,),
        in_specs=[_deg_spec(), _part_a(HID), _part_b(HID), _rows(HID),
                  _full((1, HID)), _full((HID, C_OUT))],
        out_specs=_rows(C_OUT),
        out_shape=jax.ShapeDtypeStruct((NPAD, C_OUT), jnp.bfloat16),
    )(deg_parts, s1, s1, y1, b1.reshape(1, HID), W2)

    s2 = _spmm_out(y2, src_p, dst_p, zeros_out)

    out = pl.pallas_call(
        _out_body,
        grid=(GRID,),
        in_specs=[_deg_spec(), _part_a(C_OUT), _part_b(C_OUT), _rows(C_OUT),
                  _full((1, C_OUT))],
        out_specs=_rows(C_OUT),
        out_shape=jax.ShapeDtypeStruct((NPAD, C_OUT), jnp.float32),
    )(deg_parts, s2, s2, y2, b2.reshape(1, C_OUT))

    return out[:N]


# speedup vs baseline: 45.2243x; 1.0408x over previous
import functools

import jax
import jax.numpy as jnp
from jax import lax
from jax.experimental import pallas as pl
from jax.experimental.pallas import tpu as pltpu
from jax.experimental.pallas import tpu_sc as plsc

N = 10000
E = 320000
F_IN = 128
HID = 128
C_OUT = 64

NPAD = 10240
NW = 32
CW = 128
NCHUNK = 80
EPT = NCHUNK * CW
EPAD = NW * EPT
RPT = NPAD // 16
BLK = 512
GRID = NPAD // BLK
NRING = 8
LAG = 6

_mesh = plsc.VectorSubcoreMesh(core_axis_name="c", subcore_axis_name="s")


@functools.partial(
    pl.kernel,
    out_type=jax.ShapeDtypeStruct((NW, NPAD), jnp.float32),
    mesh=_mesh,
    scratch_types=[
        pltpu.VMEM((EPT,), jnp.int32),
        pltpu.VMEM((NPAD,), jnp.float32),
    ],
    compiler_params=pltpu.CompilerParams(needs_layout_passes=False),
)
def _deg_kernel(dst_hbm, out_hbm, dst_v, deg_v):
    c = lax.axis_index("c")
    s = lax.axis_index("s")
    wid = s * 2 + c
    pltpu.sync_copy(dst_hbm.at[wid], dst_v)
    zeros = jnp.zeros((16,), jnp.float32)
    ones = jnp.ones((16,), jnp.float32)

    def zero_body(i, _):
        for u in range(4):
            deg_v[pl.ds((i * 4 + u) * 16, 16)] = zeros
        return 0

    lax.fori_loop(0, NPAD // 64, zero_body, 0)

    def body(i, _):
        idx = dst_v[pl.ds(i * 16, 16)]
        plsc.addupdate_scatter(deg_v, [idx], ones)
        return 0

    lax.fori_loop(0, EPT // 16, body, 0)
    pltpu.sync_copy(deg_v, out_hbm.at[wid])


def _make_spmm(d):
    @functools.partial(
        pl.kernel,
        out_type=jax.ShapeDtypeStruct((2, NPAD, d), jnp.bfloat16),
        mesh=_mesh,
        scratch_types=[
            pltpu.VMEM((EPT,), jnp.int32),
            pltpu.VMEM((NCHUNK, CW), jnp.int32),
            [pltpu.VMEM((CW, d), jnp.bfloat16) for _ in range(NRING)],
            pltpu.VMEM_SHARED((NPAD, d), jnp.bfloat16),
            [pltpu.SemaphoreType.DMA for _ in range(NRING)],
            [pltpu.SemaphoreType.DMA for _ in range(NRING)],
        ],
        compiler_params=pltpu.CompilerParams(use_tc_tiling_on_sc=False),
    )
    def spmm(y_hbm, src_hbm, dst_hbm, zeros_hbm, out_hbm,
             src_v, dst_v, bufs, acc_sh, gsems, ssems):
        c = lax.axis_index("c")
        s = lax.axis_index("s")
        wid = s * 2 + c
        pltpu.sync_copy(zeros_hbm.at[pl.ds(s * RPT, RPT)],
                        acc_sh.at[pl.ds(s * RPT, RPT)])
        pltpu.sync_copy(src_hbm.at[wid], src_v)
        pltpu.sync_copy(dst_hbm.at[wid], dst_v)

        def src_at(j):
            return y_hbm.at[src_v.at[pl.ds(j * CW, CW)]]

        def gather(j, b):
            pltpu.async_copy(src_at(j), bufs[b], gsems[b])

        def wait_gather(j, b):
            pltpu.make_async_copy(src_at(j), bufs[b], gsems[b]).wait()

        def scatter(j, b):
            pltpu.async_copy(bufs[b], acc_sh.at[dst_v.at[j]], ssems[b],
                             add=True)

        def wait_scatter(j, b):
            pltpu.make_async_copy(bufs[b], acc_sh.at[dst_v.at[j]],
                                  ssems[b]).wait()

        plsc.subcore_barrier()
        for b in range(LAG):
            gather(b, b)

        for jj in range(NRING):
            b = jj % NRING
            wait_gather(jj, b)
            scatter(jj, b)
            bg = (jj + LAG) % NRING
            if jj + LAG >= NRING:
                wait_scatter(jj + LAG - NRING, bg)
            gather(jj + LAG, bg)

        def body(i, _):
            for b in range(NRING):
                jj = i * NRING + b
                wait_gather(jj, b)
                scatter(jj, b)
                bg = (b + LAG) % NRING
                wait_scatter(jj + LAG - NRING, bg)
                gather(jj + LAG, bg)
            return 0

        lax.fori_loop(1, NCHUNK // NRING - 1, body, 0)

        for b in range(NRING):
            jj = NCHUNK - NRING + b
            wait_gather(jj, b)
            scatter(jj, b)
            if jj + LAG < NCHUNK:
                bg = (b + LAG) % NRING
                wait_scatter(jj + LAG - NRING, bg)
                gather(jj + LAG, bg)
        for b in range(NRING):
            wait_scatter(NCHUNK - NRING + b, b)

        plsc.subcore_barrier()
        pltpu.sync_copy(acc_sh.at[pl.ds(s * RPT, RPT)],
                        out_hbm.at[c, pl.ds(s * RPT, RPT)])

    return spmm


_spmm_hid = _make_spmm(HID)
_spmm_out = _make_spmm(C_OUT)


def _dinv_of(deg_blk):
    deg = jnp.sum(deg_blk, axis=0)
    return lax.rsqrt(deg + 1.0)


def _y1_body(deg_ref, x_ref, w1_ref, y_ref):
    i = pl.program_id(0)
    dinv = _dinv_of(deg_ref[...])
    xw = jnp.dot(x_ref[...], w1_ref[...], preferred_element_type=jnp.float32)
    rowid = lax.broadcasted_iota(jnp.int32, (BLK, HID), 0) + i * BLK
    y = jnp.where(rowid < N, xw * dinv[:, None], 0.0)
    y_ref[...] = y.astype(jnp.bfloat16)


def _y2_body(deg_ref, s1a_ref, s1b_ref, y1_ref, b1_ref, w2_ref, y2_ref):
    i = pl.program_id(0)
    dinv = _dinv_of(deg_ref[...])
    agg = (s1a_ref[0].astype(jnp.float32) + s1b_ref[0].astype(jnp.float32)
           + y1_ref[...].astype(jnp.float32))
    h = jnp.maximum(agg * dinv[:, None] + b1_ref[...], 0.0)
    rowid = lax.broadcasted_iota(jnp.int32, (BLK, HID), 0) + i * BLK
    h = jnp.where(rowid < N, h, 0.0)
    xw2 = jnp.dot(h, w2_ref[...], preferred_element_type=jnp.float32)
    y2_ref[...] = (xw2 * dinv[:, None]).astype(jnp.bfloat16)


def _out_body(deg_ref, s2a_ref, s2b_ref, y2_ref, b2_ref, out_ref):
    dinv = _dinv_of(deg_ref[...])
    agg = (s2a_ref[0].astype(jnp.float32) + s2b_ref[0].astype(jnp.float32)
           + y2_ref[...].astype(jnp.float32))
    out_ref[...] = agg * dinv[:, None] + b2_ref[...]


def _deg_spec():
    return pl.BlockSpec((NW, BLK), lambda i: (0, i))


def _rows(d):
    return pl.BlockSpec((BLK, d), lambda i: (i, 0))


def _part_a(d):
    return pl.BlockSpec((1, BLK, d), lambda i: (0, i, 0))


def _part_b(d):
    return pl.BlockSpec((1, BLK, d), lambda i: (1, i, 0))


def _full(shape):
    return pl.BlockSpec(shape, lambda i: (0,) * len(shape))


def kernel(x, edge_index, W1, b1, W2, b2):
    src = edge_index[0]
    dst = edge_index[1]
    pad_idx = N + (jnp.arange(EPAD - E, dtype=jnp.int32) % (NPAD - N))
    src_p = jnp.concatenate([src, pad_idx]).reshape(NW, EPT)
    dst_p = jnp.concatenate([dst, pad_idx]).reshape(NW, NCHUNK, CW)
    zeros_hid = jnp.zeros((NPAD, HID), jnp.bfloat16)
    zeros_out = jnp.zeros((NPAD, C_OUT), jnp.bfloat16)

    x_p = jnp.concatenate(
        [x, jnp.zeros((NPAD - N, F_IN), jnp.float32)], axis=0)

    deg_parts = _deg_kernel(dst_p.reshape(NW, EPT))

    y1 = pl.pallas_call(
        _y1_body,
        grid=(GRID,),
        in_specs=[_deg_spec(), _rows(F_IN), _full((F_IN, HID))],
        out_specs=_rows(HID),
        out_shape=jax.ShapeDtypeStruct((NPAD, HID), jnp.bfloat16),
    )(deg_parts, x_p, W1)

    s1 = _spmm_hid(y1, src_p, dst_p, zeros_hid)

    y2 = pl.pallas_call(
        _y2_body,
        grid=(GRID,),
        in_specs=[_deg_spec(), _part_a(HID), _part_b(HID), _rows(HID),
                  _full((1, HID)), _full((HID, C_OUT))],
        out_specs=_rows(C_OUT),
        out_shape=jax.ShapeDtypeStruct((NPAD, C_OUT), jnp.bfloat16),
    )(deg_parts, s1, s1, y1, b1.reshape(1, HID), W2)

    s2 = _spmm_out(y2, src_p, dst_p, zeros_out)

    out = pl.pallas_call(
        _out_body,
        grid=(GRID,),
        in_specs=[_deg_spec(), _part_a(C_OUT), _part_b(C_OUT), _rows(C_OUT),
                  _full((1, C_OUT))],
        out_specs=_rows(C_OUT),
        out_shape=jax.ShapeDtypeStruct((NPAD, C_OUT), jnp.float32),
    )(deg_parts, s2, s2, y2, b2.reshape(1, C_OUT))

    return out[:N]


# speedup vs baseline: 45.2479x; 1.0005x over previous
import functools

import jax
import jax.numpy as jnp
from jax import lax
from jax.experimental import pallas as pl
from jax.experimental.pallas import tpu as pltpu
from jax.experimental.pallas import tpu_sc as plsc

N = 10000
E = 320000
F_IN = 128
HID = 128
C_OUT = 64

NPAD = 10240
NW = 32
CW = 128
NCHUNK = 80
EPT = NCHUNK * CW
EPAD = NW * EPT
RPT = NPAD // 16
BLK = 512
GRID = NPAD // BLK
NRING = 8
LAG = 7

_mesh = plsc.VectorSubcoreMesh(core_axis_name="c", subcore_axis_name="s")


@functools.partial(
    pl.kernel,
    out_type=jax.ShapeDtypeStruct((NW, NPAD), jnp.float32),
    mesh=_mesh,
    scratch_types=[
        pltpu.VMEM((EPT,), jnp.int32),
        pltpu.VMEM((NPAD,), jnp.float32),
    ],
    compiler_params=pltpu.CompilerParams(needs_layout_passes=False),
)
def _deg_kernel(dst_hbm, out_hbm, dst_v, deg_v):
    c = lax.axis_index("c")
    s = lax.axis_index("s")
    wid = s * 2 + c
    pltpu.sync_copy(dst_hbm.at[wid], dst_v)
    zeros = jnp.zeros((16,), jnp.float32)
    ones = jnp.ones((16,), jnp.float32)

    def zero_body(i, _):
        for u in range(4):
            deg_v[pl.ds((i * 4 + u) * 16, 16)] = zeros
        return 0

    lax.fori_loop(0, NPAD // 64, zero_body, 0)

    def body(i, _):
        idx = dst_v[pl.ds(i * 16, 16)]
        plsc.addupdate_scatter(deg_v, [idx], ones)
        return 0

    lax.fori_loop(0, EPT // 16, body, 0)
    pltpu.sync_copy(deg_v, out_hbm.at[wid])


def _make_spmm(d):
    @functools.partial(
        pl.kernel,
        out_type=jax.ShapeDtypeStruct((2, NPAD, d), jnp.bfloat16),
        mesh=_mesh,
        scratch_types=[
            pltpu.VMEM((EPT,), jnp.int32),
            pltpu.VMEM((NCHUNK, CW), jnp.int32),
            [pltpu.VMEM((CW, d), jnp.bfloat16) for _ in range(NRING)],
            pltpu.VMEM_SHARED((NPAD, d), jnp.bfloat16),
            [pltpu.SemaphoreType.DMA for _ in range(NRING)],
            [pltpu.SemaphoreType.DMA for _ in range(NRING)],
        ],
        compiler_params=pltpu.CompilerParams(use_tc_tiling_on_sc=False),
    )
    def spmm(y_hbm, src_hbm, dst_hbm, zeros_hbm, out_hbm,
             src_v, dst_v, bufs, acc_sh, gsems, ssems):
        c = lax.axis_index("c")
        s = lax.axis_index("s")
        wid = s * 2 + c
        pltpu.sync_copy(zeros_hbm.at[pl.ds(s * RPT, RPT)],
                        acc_sh.at[pl.ds(s * RPT, RPT)])
        pltpu.sync_copy(src_hbm.at[wid], src_v)
        pltpu.sync_copy(dst_hbm.at[wid], dst_v)

        def src_at(j):
            return y_hbm.at[src_v.at[pl.ds(j * CW, CW)]]

        def gather(j, b):
            pltpu.async_copy(src_at(j), bufs[b], gsems[b])

        def wait_gather(j, b):
            pltpu.make_async_copy(src_at(j), bufs[b], gsems[b]).wait()

        def scatter(j, b):
            pltpu.async_copy(bufs[b], acc_sh.at[dst_v.at[j]], ssems[b],
                             add=True)

        def wait_scatter(j, b):
            pltpu.make_async_copy(bufs[b], acc_sh.at[dst_v.at[j]],
                                  ssems[b]).wait()

        plsc.subcore_barrier()
        for b in range(LAG):
            gather(b, b)

        for jj in range(NRING):
            b = jj % NRING
            wait_gather(jj, b)
            scatter(jj, b)
            bg = (jj + LAG) % NRING
            if jj + LAG >= NRING:
                wait_scatter(jj + LAG - NRING, bg)
            gather(jj + LAG, bg)

        def body(i, _):
            for b in range(NRING):
                jj = i * NRING + b
                wait_gather(jj, b)
                scatter(jj, b)
                bg = (b + LAG) % NRING
                wait_scatter(jj + LAG - NRING, bg)
                gather(jj + LAG, bg)
            return 0

        lax.fori_loop(1, NCHUNK // NRING - 1, body, 0)

        for b in range(NRING):
            jj = NCHUNK - NRING + b
            wait_gather(jj, b)
            scatter(jj, b)
            if jj + LAG < NCHUNK:
                bg = (b + LAG) % NRING
                wait_scatter(jj + LAG - NRING, bg)
                gather(jj + LAG, bg)
        for b in range(NRING):
            wait_scatter(NCHUNK - NRING + b, b)

        plsc.subcore_barrier()
        pltpu.sync_copy(acc_sh.at[pl.ds(s * RPT, RPT)],
                        out_hbm.at[c, pl.ds(s * RPT, RPT)])

    return spmm


_spmm_hid = _make_spmm(HID)
_spmm_out = _make_spmm(C_OUT)


def _dinv_of(deg_blk):
    deg = jnp.sum(deg_blk, axis=0)
    return lax.rsqrt(deg + 1.0)


def _y1_body(deg_ref, x_ref, w1_ref, y_ref):
    i = pl.program_id(0)
    dinv = _dinv_of(deg_ref[...])
    xw = jnp.dot(x_ref[...], w1_ref[...], preferred_element_type=jnp.float32)
    rowid = lax.broadcasted_iota(jnp.int32, (BLK, HID), 0) + i * BLK
    y = jnp.where(rowid < N, xw * dinv[:, None], 0.0)
    y_ref[...] = y.astype(jnp.bfloat16)


def _y2_body(deg_ref, s1a_ref, s1b_ref, y1_ref, b1_ref, w2_ref, y2_ref):
    i = pl.program_id(0)
    dinv = _dinv_of(deg_ref[...])
    agg = (s1a_ref[0].astype(jnp.float32) + s1b_ref[0].astype(jnp.float32)
           + y1_ref[...].astype(jnp.float32))
    h = jnp.maximum(agg * dinv[:, None] + b1_ref[...], 0.0)
    rowid = lax.broadcasted_iota(jnp.int32, (BLK, HID), 0) + i * BLK
    h = jnp.where(rowid < N, h, 0.0)
    xw2 = jnp.dot(h, w2_ref[...], preferred_element_type=jnp.float32)
    y2_ref[...] = (xw2 * dinv[:, None]).astype(jnp.bfloat16)


def _out_body(deg_ref, s2a_ref, s2b_ref, y2_ref, b2_ref, out_ref):
    dinv = _dinv_of(deg_ref[...])
    agg = (s2a_ref[0].astype(jnp.float32) + s2b_ref[0].astype(jnp.float32)
           + y2_ref[...].astype(jnp.float32))
    out_ref[...] = agg * dinv[:, None] + b2_ref[...]


def _deg_spec():
    return pl.BlockSpec((NW, BLK), lambda i: (0, i))


def _rows(d):
    return pl.BlockSpec((BLK, d), lambda i: (i, 0))


def _part_a(d):
    return pl.BlockSpec((1, BLK, d), lambda i: (0, i, 0))


def _part_b(d):
    return pl.BlockSpec((1, BLK, d), lambda i: (1, i, 0))


def _full(shape):
    return pl.BlockSpec(shape, lambda i: (0,) * len(shape))


def kernel(x, edge_index, W1, b1, W2, b2):
    src = edge_index[0]
    dst = edge_index[1]
    pad_idx = N + (jnp.arange(EPAD - E, dtype=jnp.int32) % (NPAD - N))
    src_p = jnp.concatenate([src, pad_idx]).reshape(NW, EPT)
    dst_p = jnp.concatenate([dst, pad_idx]).reshape(NW, NCHUNK, CW)
    zeros_hid = jnp.zeros((NPAD, HID), jnp.bfloat16)
    zeros_out = jnp.zeros((NPAD, C_OUT), jnp.bfloat16)

    x_p = jnp.concatenate(
        [x, jnp.zeros((NPAD - N, F_IN), jnp.float32)], axis=0)

    deg_parts = _deg_kernel(dst_p.reshape(NW, EPT))

    y1 = pl.pallas_call(
        _y1_body,
        grid=(GRID,),
        in_specs=[_deg_spec(), _rows(F_IN), _full((F_IN, HID))],
        out_specs=_rows(HID),
        out_shape=jax.ShapeDtypeStruct((NPAD, HID), jnp.bfloat16),
    )(deg_parts, x_p, W1)

    s1 = _spmm_hid(y1, src_p, dst_p, zeros_hid)

    y2 = pl.pallas_call(
        _y2_body,
        grid=(GRID,),
        in_specs=[_deg_spec(), _part_a(HID), _part_b(HID), _rows(HID),
                  _full((1, HID)), _full((HID, C_OUT))],
        out_specs=_rows(C_OUT),
        out_shape=jax.ShapeDtypeStruct((NPAD, C_OUT), jnp.bfloat16),
    )(deg_parts, s1, s1, y1, b1.reshape(1, HID), W2)

    s2 = _spmm_out(y2, src_p, dst_p, zeros_out)

    out = pl.pallas_call(
        _out_body,
        grid=(GRID,),
        in_specs=[_deg_spec(), _part_a(C_OUT), _part_b(C_OUT), _rows(C_OUT),
                  _full((1, C_OUT))],
        out_specs=_rows(C_OUT),
        out_shape=jax.ShapeDtypeStruct((NPAD, C_OUT), jnp.float32),
    )(deg_parts, s2, s2, y2, b2.reshape(1, C_OUT))

    return out[:N]
